# baseline (speedup 1.0000x reference)
import functools

import jax
import jax.numpy as jnp
from jax import lax
from jax.experimental import pallas as pl
from jax.experimental.pallas import tpu as pltpu
from jax.experimental.pallas import tpu_sc as plsc

_WEIGHT1 = 1.0
_WEIGHT2 = 0.5
_SIGMA1 = 1.0
_SIGMA2 = 2.0
_WEIGHT_MARGIN = 0.8

_NC = 2
_NS = 16
_L = 16
_CB = 128
_GCHUNK = 128


def _sc_loss_partials(preds, targets, margin, score, B, C):
    nw = _NC * _NS
    bpw = B // nw
    nv = bpw // _L
    nb = (C + _CB - 1) // _CB
    tail_w = C - (nb - 1) * _CB
    bstride = bpw + _L
    max_chunks = (bpw + _GCHUNK - 1) // _GCHUNK

    mesh = plsc.VectorSubcoreMesh(core_axis_name="c", subcore_axis_name="s")

    @functools.partial(
        pl.kernel,
        mesh=mesh,
        out_type=jax.ShapeDtypeStruct((nw * _L,), jnp.float32),
        scratch_types=[
            pltpu.VMEM((bpw,), jnp.int32),
            pltpu.VMEM((bpw,), jnp.float32),
            pltpu.VMEM((bpw,), jnp.float32),
            pltpu.VMEM((bpw,), jnp.float32),
            pltpu.VMEM((nb * bstride,), jnp.int32),
            pltpu.VMEM((nb * bstride,), jnp.int32),
            pltpu.VMEM((nb * bstride,), jnp.float32),
            pltpu.VMEM((_GCHUNK, _CB), jnp.float32),
            pltpu.VMEM((_L,), jnp.float32),
            pltpu.VMEM((_L,), jnp.float32),
            pltpu.SMEM((nb,), jnp.int32),
            pltpu.SemaphoreType.DMA,
        ],
    )
    def body(preds_hbm, tgt_hbm, mar_hbm, sco_hbm, out_hbm,
             t_v, m_v, s_v, w_v, bkt_r, bkt_t, bkt_w, lines_v,
             acc_v, p_v, cnt_s, sem):
        wid = lax.axis_index("s") * _NC + lax.axis_index("c")
        base = wid * bpw
        lane = lax.iota(jnp.int32, _L)
        zero = jnp.zeros((_L,), jnp.float32)

        pltpu.sync_copy(tgt_hbm.at[pl.ds(base, bpw)], t_v)
        pltpu.sync_copy(mar_hbm.at[pl.ds(base, bpw)], m_v)
        pltpu.sync_copy(sco_hbm.at[pl.ds(base, bpw)], s_v)

        zeros_i = jnp.zeros((_L,), jnp.int32)

        def clear(j, carry):
            bkt_r[pl.ds(j * _L, _L)] = zeros_i
            return carry

        lax.fori_loop(0, nb * bstride // _L, clear, 0, unroll=4)

        def weigh(j, carry):
            sl = pl.ds(j * _L, _L)
            m = m_v[sl]
            m2 = m * m
            w_v[sl] = (
                jnp.where(m > 0, _WEIGHT1 * jnp.exp(-_SIGMA1 * m2), 0.0)
                + jnp.where(m < 0, _WEIGHT2 * jnp.exp(-_SIGMA2 * m2), 0.0))
            return carry

        lax.fori_loop(0, nv, weigh, 0, unroll=2)

        for b in range(nb):
            cnt_s[b] = 0
        acc_v[...] = zero

        def route(j, carry):
            t16 = t_v[pl.ds(j * _L, _L)]
            w16 = w_v[pl.ds(j * _L, _L)]
            for i in range(_L):
                tt = t16[i]
                wg = w16[i]
                b = lax.shift_right_logical(tt, 7)
                cnt = cnt_s[b]
                addr = b * bstride + cnt
                wb = (addr // _L) * _L
                lm = lane == addr - wb
                rid = base + j * _L + i
                bkt_r[pl.ds(wb, _L)] = jnp.where(lm, rid, bkt_r[pl.ds(wb, _L)])
                bkt_t[pl.ds(wb, _L)] = jnp.where(lm, tt, bkt_t[pl.ds(wb, _L)])
                bkt_w[pl.ds(wb, _L)] = jnp.where(lm, wg, bkt_w[pl.ds(wb, _L)])
                cnt_s[b] = cnt + 1
            return carry

        lax.fori_loop(0, nv, route, 0)

        def bucket_step(b, carry):
            col = b * _CB
            n_rows = cnt_s[b]
            for c in range(max_chunks):

                @pl.when(n_rows > c * _GCHUNK)
                def _(col=col, n_rows=n_rows, c=c):
                    cbase = b * bstride + c * _GCHUNK
                    idx_win = bkt_r.at[pl.ds(cbase, _GCHUNK)]
                    pltpu.async_copy(
                        preds_hbm.at[idx_win, pl.ds(col, _CB)],
                        lines_v, sem,
                    ).wait()

                    def pick(g, acc):
                        gbase = cbase + g * _L
                        tt16 = bkt_t[pl.ds(gbase, _L)]
                        wg16 = bkt_w[pl.ds(gbase, _L)]
                        off16 = jnp.clip(tt16 - col, 0, _CB - 1)
                        valid = (c * _GCHUNK + g * _L + lane) < n_rows
                        wgm16 = jnp.where(valid, wg16, 0.0)
                        for i in range(_L):
                            off = off16[i]
                            wgt = wgm16[i]
                            seg = lines_v[g * _L + i,
                                          pl.ds((off // _L) * _L, _L)]
                            acc = acc + jnp.where(
                                lane == off % _L, wgt * seg, 0.0)
                        return acc

                    acc_v[...] += lax.fori_loop(
                        0, _GCHUNK // _L, pick, zero)

            return carry

        lax.fori_loop(0, nb, bucket_step, 0)

        def sstep(j, carry):
            asum, acnt = carry
            s16 = s_v[pl.ds(j * _L, _L)]
            neg = s16 < 0
            return (asum + jnp.where(neg, s16, 0.0),
                    acnt + jnp.where(neg, 1.0, 0.0))

        asum, acnt = lax.fori_loop(0, nv, sstep, (zero, zero), unroll=2)

        def lane_total(v):
            for sh in (8, 4, 2, 1):
                v = v + v.at[lane ^ sh].get(mode="promise_in_bounds")
            return v

        p_v[...] = (jnp.where(lane == 0, lane_total(acc_v[...]), 0.0)
                    + jnp.where(lane == 1, lane_total(asum), 0.0)
                    + jnp.where(lane == 2, lane_total(acnt), 0.0))
        pltpu.sync_copy(p_v, out_hbm.at[pl.ds(wid * _L, _L)])

    return body(preds, targets, margin, score)


def kernel(preds, targets, margin, score):
    B, C = preds.shape
    parts = _sc_loss_partials(preds, targets, margin, score, B, C)
    p = parts.reshape(_NC * _NS, _L)
    w_loss = p[:, 0].sum()
    neg_sum = p[:, 1].sum()
    neg_cnt = p[:, 2].sum()
    return -w_loss / B + _WEIGHT_MARGIN * (neg_sum / neg_cnt)


# speedup vs baseline: 3.7237x; 3.7237x over previous
import jax
import jax.numpy as jnp
from jax import lax
from jax.experimental import pallas as pl
from jax.experimental.pallas import tpu as pltpu

_WEIGHT1 = 1.0
_WEIGHT2 = 0.5
_SIGMA1 = 1.0
_SIGMA2 = 2.0
_WEIGHT_MARGIN = 0.8

_BR = 2048


def _body(preds_ref, tgt_ref, mar_ref, sco_ref, out_ref, acc_ref, sc_ref):
    i = pl.program_id(0)
    n_steps = pl.num_programs(0)

    @pl.when(i == 0)
    def _():
        acc_ref[...] = jnp.zeros_like(acc_ref)
        sc_ref[...] = jnp.zeros_like(sc_ref)

    sl = pl.ds(i * _BR, _BR)
    t = tgt_ref[sl]
    m = mar_ref[sl]
    s = sco_ref[sl]

    m2 = m * m
    w = (jnp.where(m > 0, _WEIGHT1 * jnp.exp(-_SIGMA1 * m2), 0.0)
         + jnp.where(m < 0, _WEIGHT2 * jnp.exp(-_SIGMA2 * m2), 0.0))

    cols = lax.broadcasted_iota(jnp.int32, preds_ref.shape, 1)
    contrib = jnp.where(cols == t[:, None], w[:, None] * preds_ref[...], 0.0)
    acc_ref[...] += jnp.sum(
        contrib.reshape(_BR // 8, 8, preds_ref.shape[1]), axis=0)

    neg = s < 0
    s_neg = jnp.sum(jnp.where(neg, s, 0.0))
    c_neg = jnp.sum(jnp.where(neg, 1.0, 0.0))
    r = lax.broadcasted_iota(jnp.int32, sc_ref.shape, 0)
    c = lax.broadcasted_iota(jnp.int32, sc_ref.shape, 1)
    first = c == 0
    sc_ref[...] += (jnp.where((r == 1) & first, s_neg, 0.0)
                    + jnp.where((r == 2) & first, c_neg, 0.0))

    @pl.when(i == n_steps - 1)
    def _():
        w_loss = jnp.sum(acc_ref[...])
        out_ref[...] = sc_ref[...] + jnp.where((r == 0) & first, w_loss, 0.0)


def kernel(preds, targets, margin, score):
    B, C = preds.shape
    grid = B // _BR
    out = pl.pallas_call(
        _body,
        grid=(grid,),
        in_specs=[
            pl.BlockSpec((_BR, C), lambda i: (i, 0)),
            pl.BlockSpec((B,), lambda i: (0,)),
            pl.BlockSpec((B,), lambda i: (0,)),
            pl.BlockSpec((B,), lambda i: (0,)),
        ],
        out_specs=pl.BlockSpec((8, 128), lambda i: (0, 0)),
        out_shape=jax.ShapeDtypeStruct((8, 128), jnp.float32),
        scratch_shapes=[
            pltpu.VMEM((8, C), jnp.float32),
            pltpu.VMEM((8, 128), jnp.float32),
        ],
        compiler_params=pltpu.CompilerParams(
            dimension_semantics=("arbitrary",),
        ),
    )(preds, targets, margin, score)
    return -out[0, 0] / B + _WEIGHT_MARGIN * (out[1, 0] / out[2, 0])


# speedup vs baseline: 3.8589x; 1.0363x over previous
import jax
import jax.numpy as jnp
from jax import lax
from jax.experimental import pallas as pl
from jax.experimental.pallas import tpu as pltpu

_WEIGHT1 = 1.0
_WEIGHT2 = 0.5
_SIGMA1 = 1.0
_SIGMA2 = 2.0
_WEIGHT_MARGIN = 0.8

_BR = 2048


def _body(preds_ref, tgt_ref, mar_ref, sco_ref, out_ref, acc_ref, sc_ref):
    i = pl.program_id(0)
    n_steps = pl.num_programs(0)

    @pl.when(i == 0)
    def _():
        acc_ref[...] = jnp.zeros_like(acc_ref)
        sc_ref[...] = jnp.zeros_like(sc_ref)

    sl = pl.ds(i * _BR, _BR)
    t = tgt_ref[sl]
    m = mar_ref[sl]
    s = sco_ref[sl]

    m2 = m * m
    w = (jnp.where(m > 0, _WEIGHT1 * jnp.exp(-_SIGMA1 * m2), 0.0)
         + jnp.where(m < 0, _WEIGHT2 * jnp.exp(-_SIGMA2 * m2), 0.0))

    cols = lax.broadcasted_iota(jnp.int32, preds_ref.shape, 1)
    contrib = jnp.where(cols == t[:, None], w[:, None] * preds_ref[...], 0.0)
    acc_ref[...] += jnp.sum(
        contrib.reshape(_BR // 8, 8, preds_ref.shape[1]), axis=0)

    neg = s < 0
    s_neg = jnp.sum(jnp.where(neg, s, 0.0))
    c_neg = jnp.sum(jnp.where(neg, 1.0, 0.0))
    r = lax.broadcasted_iota(jnp.int32, sc_ref.shape, 0)
    c = lax.broadcasted_iota(jnp.int32, sc_ref.shape, 1)
    first = c == 0
    sc_ref[...] += (jnp.where((r == 1) & first, s_neg, 0.0)
                    + jnp.where((r == 2) & first, c_neg, 0.0))

    @pl.when(i == n_steps - 1)
    def _():
        w_loss = jnp.sum(acc_ref[...])
        neg_sum = jnp.sum(jnp.where((r == 1) & first, sc_ref[...], 0.0))
        neg_cnt = jnp.sum(jnp.where((r == 2) & first, sc_ref[...], 0.0))
        b_rows = _BR * n_steps
        out_ref[0] = (-w_loss / b_rows
                      + _WEIGHT_MARGIN * (neg_sum / neg_cnt))


def kernel(preds, targets, margin, score):
    B, C = preds.shape
    grid = B // _BR
    out = pl.pallas_call(
        _body,
        grid=(grid,),
        in_specs=[
            pl.BlockSpec((_BR, C), lambda i: (i, 0)),
            pl.BlockSpec((B,), lambda i: (0,)),
            pl.BlockSpec((B,), lambda i: (0,)),
            pl.BlockSpec((B,), lambda i: (0,)),
        ],
        out_specs=pl.BlockSpec(memory_space=pltpu.SMEM),
        out_shape=jax.ShapeDtypeStruct((1,), jnp.float32),
        scratch_shapes=[
            pltpu.VMEM((8, C), jnp.float32),
            pltpu.VMEM((8, 128), jnp.float32),
        ],
        compiler_params=pltpu.CompilerParams(
            dimension_semantics=("arbitrary",),
        ),
    )(preds, targets, margin, score)
    return out[0]
